# baseline (speedup 1.0000x reference)
import functools

import jax
import jax.numpy as jnp
from jax.experimental import pallas as pl
from jax.experimental.pallas import tpu as pltpu

_N = 100000
_SEG = 512
_T = 1000.0
_B = 2000
_G = _N // _B


def _body(t_ref, seg_ref, dx_ref, zx_ref, dh_ref, zh_ref, out_ref, acc_ref):
    i = pl.program_id(0)

    @pl.when(i == 0)
    def _init():
        acc_ref[...] = jnp.zeros_like(acc_ref)

    t = t_ref[0]
    t_int = jnp.round(t * _T)
    s_t = t_int * (1.0 / _T)
    s_s = (t_int - 1.0) * (1.0 / _T)
    z_t = 1e-4 + 10.0 * s_t * s_t
    z_s = 1e-4 + 10.0 * s_s * s_s
    def _inv_expm1(z):
        u = jnp.exp(z)
        d = u - 1.0
        em1 = jnp.where(d == 0.0, z, d * z / jnp.log(u))
        return 1.0 / em1

    snr_w = _inv_expm1(z_s) - _inv_expm1(z_t)

    dx = dx_ref[...] - zx_ref[...]
    dh = dh_ref[...] - zh_ref[...]
    sq = (jnp.sum(dx * dx, axis=1, keepdims=True)
          + jnp.sum(dh * dh, axis=1, keepdims=True))
    pn = snr_w * sq * jnp.where(t_int == 0.0, 0.0, 1.0)

    p_mat = jnp.concatenate([pn, jnp.ones_like(pn)], axis=1)
    ids = seg_ref[0]
    one_hot = (jnp.broadcast_to(ids, (_B, _SEG))
               == jax.lax.broadcasted_iota(jnp.int32, (_B, _SEG), 1)
               ).astype(jnp.float32)
    acc_ref[...] += jax.lax.dot_general(
        one_hot, p_mat, (((0,), (0,)), ((), ())),
        preferred_element_type=jnp.float32)

    @pl.when(i == _G - 1)
    def _fini():
        seg_sum = acc_ref[:, 0:1]
        seg_cnt = acc_ref[:, 1:2]
        loss = seg_sum / jnp.maximum(seg_cnt, 1.0)
        out_ref[...] = jnp.sum(loss, axis=0, keepdims=True) * (1.0 / _SEG)


@jax.jit
def kernel(t, dx_t, dh_t, z_x, z_h, x, h, segment_ids):
    del x, h
    t3 = t.reshape(_G, _B, 1)
    seg3 = segment_ids.astype(jnp.int32).reshape(_G, _B, 1)
    out = pl.pallas_call(
        _body,
        grid=(_G,),
        in_specs=[
            pl.BlockSpec((1, _B, 1), lambda i: (i, 0, 0)),
            pl.BlockSpec((1, _B, 1), lambda i: (i, 0, 0)),
            pl.BlockSpec((_B, 3), lambda i: (i, 0)),
            pl.BlockSpec((_B, 3), lambda i: (i, 0)),
            pl.BlockSpec((_B, 128), lambda i: (i, 0)),
            pl.BlockSpec((_B, 128), lambda i: (i, 0)),
        ],
        out_specs=pl.BlockSpec((1, 1), lambda i: (0, 0)),
        out_shape=jax.ShapeDtypeStruct((1, 1), jnp.float32),
        scratch_shapes=[pltpu.VMEM((_SEG, 2), jnp.float32)],
    )(t3, seg3, dx_t, z_x, dh_t, z_h)
    return out[0, 0]


# speedup vs baseline: 2.1623x; 2.1623x over previous
import jax
import jax.numpy as jnp
from jax.experimental import pallas as pl
from jax.experimental.pallas import tpu as pltpu

_N = 100000
_SEG = 512
_T = 1000.0
_B = 2000
_G = _N // _B


def _inv_expm1(z):
    u = jnp.exp(z)
    d = u - 1.0
    em1 = jnp.where(d == 0.0, z, d * z / jnp.log(u))
    return 1.0 / em1


def _body(t_ref, seg_ref, dx_ref, zx_ref, dh_ref, zh_ref, out_ref, acc_ref):
    i = pl.program_id(0)

    @pl.when(i == 0)
    def _init():
        acc_ref[...] = jnp.zeros_like(acc_ref)

    t = t_ref[0]
    t_int = jnp.round(t * _T)
    s_t = t_int * (1.0 / _T)
    s_s = (t_int - 1.0) * (1.0 / _T)
    z_t = 1e-4 + 10.0 * s_t * s_t
    z_s = 1e-4 + 10.0 * s_s * s_s
    snr_w = _inv_expm1(z_s) - _inv_expm1(z_t)
    g = jnp.where(t_int == 0.0, 0.0, snr_w)

    dx = dx_ref[...] - zx_ref[...]
    dh = dh_ref[...] - zh_ref[...]
    ones_x = jnp.ones((1, 3), jnp.float32)
    ones_h = jnp.ones((1, 128), jnp.float32)
    dnums = (((1,), (1,)), ((), ()))
    sq = (jax.lax.dot_general(ones_x, dx * dx, dnums,
                              preferred_element_type=jnp.float32)
          + jax.lax.dot_general(ones_h, dh * dh, dnums,
                                preferred_element_type=jnp.float32))
    pn = g * sq

    p2 = jnp.concatenate([pn, jnp.ones_like(pn)], axis=0)
    ids = seg_ref[0]
    one_hot = (jnp.broadcast_to(ids, (_SEG, _B))
               == jax.lax.broadcasted_iota(jnp.int32, (_SEG, _B), 0)
               ).astype(jnp.float32)
    acc_ref[...] += jax.lax.dot_general(
        one_hot, p2, dnums, preferred_element_type=jnp.float32)

    @pl.when(i == _G - 1)
    def _fini():
        seg_sum = acc_ref[:, 0:1]
        seg_cnt = acc_ref[:, 1:2]
        loss = seg_sum / jnp.maximum(seg_cnt, 1.0)
        out_ref[...] = jnp.sum(loss, axis=0, keepdims=True) * (1.0 / _SEG)


@jax.jit
def kernel(t, dx_t, dh_t, z_x, z_h, x, h, segment_ids):
    del x, h
    t3 = t.reshape(_G, 1, _B)
    seg3 = segment_ids.astype(jnp.int32).reshape(_G, 1, _B)
    out = pl.pallas_call(
        _body,
        grid=(_G,),
        in_specs=[
            pl.BlockSpec((1, 1, _B), lambda i: (i, 0, 0)),
            pl.BlockSpec((1, 1, _B), lambda i: (i, 0, 0)),
            pl.BlockSpec((_B, 3), lambda i: (i, 0)),
            pl.BlockSpec((_B, 3), lambda i: (i, 0)),
            pl.BlockSpec((_B, 128), lambda i: (i, 0)),
            pl.BlockSpec((_B, 128), lambda i: (i, 0)),
        ],
        out_specs=pl.BlockSpec((1, 1), lambda i: (0, 0)),
        out_shape=jax.ShapeDtypeStruct((1, 1), jnp.float32),
        scratch_shapes=[pltpu.VMEM((_SEG, 2), jnp.float32)],
    )(t3, seg3, dx_t, z_x, dh_t, z_h)
    return out[0, 0]


# speedup vs baseline: 2.4608x; 1.1380x over previous
import jax
import jax.numpy as jnp
from jax.experimental import pallas as pl
from jax.experimental.pallas import tpu as pltpu

_N = 100000
_SEG = 512
_T = 1000.0
_B = 4000
_G = _N // _B


def _inv_expm1(z):
    u = jnp.exp(z)
    d = u - 1.0
    em1 = jnp.where(d == 0.0, z, d * z / jnp.log(u))
    return 1.0 / em1


def _body(t_ref, seg_ref, dx_ref, zx_ref, dh_ref, zh_ref, out_ref, acc_ref):
    i = pl.program_id(0)

    @pl.when(i == 0)
    def _init():
        acc_ref[...] = jnp.zeros_like(acc_ref)

    t = t_ref[0]
    t_int = jnp.round(t * _T)
    s_t = t_int * (1.0 / _T)
    s_s = (t_int - 1.0) * (1.0 / _T)
    z_t = 1e-4 + 10.0 * s_t * s_t
    z_s = 1e-4 + 10.0 * s_s * s_s
    snr_w = _inv_expm1(z_s) - _inv_expm1(z_t)
    g = jnp.where(t_int == 0.0, 0.0, snr_w)

    dx = dx_ref[...] - zx_ref[...]
    dh = dh_ref[...] - zh_ref[...]
    ones_x = jnp.ones((1, 3), jnp.float32)
    ones_h = jnp.ones((1, 128), jnp.float32)
    dnums = (((1,), (1,)), ((), ()))
    sq = (jax.lax.dot_general(ones_x, dx * dx, dnums,
                              preferred_element_type=jnp.float32)
          + jax.lax.dot_general(ones_h, dh * dh, dnums,
                                preferred_element_type=jnp.float32))
    pn = g * sq

    p2 = jnp.concatenate([pn, jnp.ones_like(pn)], axis=0)
    ids = seg_ref[0]
    one_hot = (jnp.broadcast_to(ids, (_SEG, _B))
               == jax.lax.broadcasted_iota(jnp.int32, (_SEG, _B), 0)
               ).astype(jnp.float32)
    acc_ref[...] += jax.lax.dot_general(
        one_hot, p2, dnums, preferred_element_type=jnp.float32)

    @pl.when(i == _G - 1)
    def _fini():
        seg_sum = acc_ref[:, 0:1]
        seg_cnt = acc_ref[:, 1:2]
        loss = seg_sum / jnp.maximum(seg_cnt, 1.0)
        out_ref[...] = jnp.sum(loss, axis=0, keepdims=True) * (1.0 / _SEG)


@jax.jit
def kernel(t, dx_t, dh_t, z_x, z_h, x, h, segment_ids):
    del x, h
    t3 = t.reshape(_G, 1, _B)
    seg3 = segment_ids.astype(jnp.int32).reshape(_G, 1, _B)
    out = pl.pallas_call(
        _body,
        grid=(_G,),
        in_specs=[
            pl.BlockSpec((1, 1, _B), lambda i: (i, 0, 0)),
            pl.BlockSpec((1, 1, _B), lambda i: (i, 0, 0)),
            pl.BlockSpec((_B, 3), lambda i: (i, 0)),
            pl.BlockSpec((_B, 3), lambda i: (i, 0)),
            pl.BlockSpec((_B, 128), lambda i: (i, 0)),
            pl.BlockSpec((_B, 128), lambda i: (i, 0)),
        ],
        out_specs=pl.BlockSpec((1, 1), lambda i: (0, 0)),
        out_shape=jax.ShapeDtypeStruct((1, 1), jnp.float32),
        scratch_shapes=[pltpu.VMEM((_SEG, 2), jnp.float32)],
    )(t3, seg3, dx_t, z_x, dh_t, z_h)
    return out[0, 0]


# speedup vs baseline: 2.5159x; 1.0224x over previous
import jax
import jax.numpy as jnp
from jax.experimental import pallas as pl
from jax.experimental.pallas import tpu as pltpu

_N = 100000
_SEG = 512
_T = 1000.0
_B = 5000
_G = _N // _B


def _inv_expm1(z):
    u = jnp.exp(z)
    d = u - 1.0
    em1 = jnp.where(d == 0.0, z, d * z / jnp.log(u))
    return 1.0 / em1


def _body(t_ref, seg_ref, dx_ref, zx_ref, dh_ref, zh_ref, out_ref, acc_ref):
    i = pl.program_id(0)

    @pl.when(i == 0)
    def _init():
        acc_ref[...] = jnp.zeros_like(acc_ref)

    t = t_ref[0]
    t_int = jnp.round(t * _T)
    s_t = t_int * (1.0 / _T)
    s_s = (t_int - 1.0) * (1.0 / _T)
    z_t = 1e-4 + 10.0 * s_t * s_t
    z_s = 1e-4 + 10.0 * s_s * s_s
    snr_w = _inv_expm1(z_s) - _inv_expm1(z_t)
    g = jnp.where(t_int == 0.0, 0.0, snr_w)

    dx = dx_ref[...] - zx_ref[...]
    dh = dh_ref[...] - zh_ref[...]
    ones_x = jnp.ones((1, 3), jnp.float32)
    ones_h = jnp.ones((1, 128), jnp.float32)
    dnums = (((1,), (1,)), ((), ()))
    sq = (jax.lax.dot_general(ones_x, dx * dx, dnums,
                              preferred_element_type=jnp.float32)
          + jax.lax.dot_general(ones_h, dh * dh, dnums,
                                preferred_element_type=jnp.float32))
    pn = g * sq

    p2 = jnp.concatenate([pn, jnp.ones_like(pn)], axis=0)
    ids = seg_ref[0]
    one_hot = (jnp.broadcast_to(ids, (_SEG, _B))
               == jax.lax.broadcasted_iota(jnp.int32, (_SEG, _B), 0)
               ).astype(jnp.float32)
    acc_ref[...] += jax.lax.dot_general(
        one_hot, p2, dnums, preferred_element_type=jnp.float32)

    @pl.when(i == _G - 1)
    def _fini():
        seg_sum = acc_ref[:, 0:1]
        seg_cnt = acc_ref[:, 1:2]
        loss = seg_sum / jnp.maximum(seg_cnt, 1.0)
        out_ref[...] = jnp.sum(loss, axis=0, keepdims=True) * (1.0 / _SEG)


@jax.jit
def kernel(t, dx_t, dh_t, z_x, z_h, x, h, segment_ids):
    del x, h
    t3 = t.reshape(_G, 1, _B)
    seg3 = segment_ids.astype(jnp.int32).reshape(_G, 1, _B)
    out = pl.pallas_call(
        _body,
        grid=(_G,),
        in_specs=[
            pl.BlockSpec((1, 1, _B), lambda i: (i, 0, 0)),
            pl.BlockSpec((1, 1, _B), lambda i: (i, 0, 0)),
            pl.BlockSpec((_B, 3), lambda i: (i, 0)),
            pl.BlockSpec((_B, 3), lambda i: (i, 0)),
            pl.BlockSpec((_B, 128), lambda i: (i, 0)),
            pl.BlockSpec((_B, 128), lambda i: (i, 0)),
        ],
        out_specs=pl.BlockSpec((1, 1), lambda i: (0, 0)),
        out_shape=jax.ShapeDtypeStruct((1, 1), jnp.float32),
        scratch_shapes=[pltpu.VMEM((_SEG, 2), jnp.float32)],
    )(t3, seg3, dx_t, z_x, dh_t, z_h)
    return out[0, 0]


# speedup vs baseline: 2.5401x; 1.0096x over previous
import jax
import jax.numpy as jnp
from jax.experimental import pallas as pl
from jax.experimental.pallas import tpu as pltpu

_N = 100000
_SEG = 512
_T = 1000.0
_B = 5000
_G = _N // (2 * _B)


def _inv_expm1(z):
    u = jnp.exp(z)
    d = u - 1.0
    em1 = jnp.where(d == 0.0, z, d * z / jnp.log(u))
    return 1.0 / em1


def _half_update(t_ref, seg_ref, dx_ref, zx_ref, dh_ref, zh_ref, acc_ref):
    t = t_ref[0]
    t_int = jnp.round(t * _T)
    s_t = t_int * (1.0 / _T)
    s_s = (t_int - 1.0) * (1.0 / _T)
    z_t = 1e-4 + 10.0 * s_t * s_t
    z_s = 1e-4 + 10.0 * s_s * s_s
    snr_w = _inv_expm1(z_s) - _inv_expm1(z_t)
    g = jnp.where(t_int == 0.0, 0.0, snr_w)

    dx = dx_ref[...] - zx_ref[...]
    dh = dh_ref[...] - zh_ref[...]
    ones_x = jnp.ones((1, 3), jnp.float32)
    ones_h = jnp.ones((1, 128), jnp.float32)
    dnums = (((1,), (1,)), ((), ()))
    sq = (jax.lax.dot_general(ones_x, dx * dx, dnums,
                              preferred_element_type=jnp.float32)
          + jax.lax.dot_general(ones_h, dh * dh, dnums,
                                preferred_element_type=jnp.float32))
    pn = g * sq

    p2 = jnp.concatenate([pn, jnp.ones_like(pn)], axis=0)
    ids = seg_ref[0]
    one_hot = (jnp.broadcast_to(ids, (_SEG, _B))
               == jax.lax.broadcasted_iota(jnp.int32, (_SEG, _B), 0)
               ).astype(jnp.float32)
    acc_ref[...] += jax.lax.dot_general(
        one_hot, p2, dnums, preferred_element_type=jnp.float32)


def _body(t_a, seg_a, dx_a, zx_a, dh_a, zh_a,
          t_b, seg_b, dx_b, zx_b, dh_b, zh_b, out_ref, acc_ref):
    i = pl.program_id(0)

    @pl.when(i == 0)
    def _init():
        acc_ref[...] = jnp.zeros_like(acc_ref)

    _half_update(t_a, seg_a, dx_a, zx_a, dh_a, zh_a, acc_ref)
    _half_update(t_b, seg_b, dx_b, zx_b, dh_b, zh_b, acc_ref)

    @pl.when(i == _G - 1)
    def _fini():
        seg_sum = acc_ref[:, 0:1]
        seg_cnt = acc_ref[:, 1:2]
        loss = seg_sum / jnp.maximum(seg_cnt, 1.0)
        out_ref[...] = jnp.sum(loss, axis=0, keepdims=True) * (1.0 / _SEG)


@jax.jit
def kernel(t, dx_t, dh_t, z_x, z_h, x, h, segment_ids):
    del x, h
    t3 = t.reshape(2 * _G, 1, _B)
    seg3 = segment_ids.astype(jnp.int32).reshape(2 * _G, 1, _B)

    def spec3(off):
        return pl.BlockSpec((1, 1, _B), lambda i, off=off: (off + i, 0, 0))

    def spec2(ncol, off):
        return pl.BlockSpec((_B, ncol), lambda i, off=off: (off + i, 0))

    def half_specs(off):
        return [spec3(off), spec3(off), spec2(3, off),
                spec2(3, off), spec2(128, off), spec2(128, off)]

    half_inputs = (t3, seg3, dx_t, z_x, dh_t, z_h)
    out = pl.pallas_call(
        _body,
        grid=(_G,),
        in_specs=half_specs(0) + half_specs(_G),
        out_specs=pl.BlockSpec((1, 1), lambda i: (0, 0)),
        out_shape=jax.ShapeDtypeStruct((1, 1), jnp.float32),
        scratch_shapes=[pltpu.VMEM((_SEG, 2), jnp.float32)],
    )(*half_inputs, *half_inputs)
    return out[0, 0]
